# baseline (speedup 1.0000x reference)
import jax
import jax.numpy as jnp
from jax import lax
from jax.experimental import pallas as pl

B, S, D, H = 64, 1024, 96, 64
R = 8


def _fused_body(x_ref, w1_ref, b1_ref, w2_ref, b2_ref, tri_ref,
                out_ref, mask_ref, ek_ref):
    i = pl.program_id(0)

    x = x_ref[...]
    x2 = x.reshape(R * S, D)
    h = jnp.dot(x2, w1_ref[...], precision=lax.Precision.DEFAULT,
                preferred_element_type=jnp.float32)
    h = jnp.maximum(h + b1_ref[...][None, :], 0.0)
    h3 = h.reshape(R, S, H).astype(jnp.bfloat16).astype(jnp.float32)
    w2 = w2_ref[...].reshape(1, 1, H).astype(jnp.bfloat16).astype(jnp.float32)
    logits = jnp.sum(h3 * w2, axis=2) + b2_ref[0]

    ek = jnp.sum(jax.nn.sigmoid(logits), axis=1, keepdims=True)
    kf = jnp.maximum(ek.astype(jnp.int32), 32).astype(jnp.float32)

    m = jnp.max(logits, axis=1, keepdims=True)
    e = jnp.exp(logits - m)
    s = e / jnp.sum(e, axis=1, keepdims=True)

    u = lax.bitcast_convert_type(s, jnp.int32)

    p = jnp.zeros((R, 1), jnp.int32)
    for bit in range(29, -1, -1):
        cand = p | (1 << bit)
        cnt = jnp.sum((u >= cand).astype(jnp.float32), axis=1, keepdims=True)
        p = jnp.where(cnt >= kf, cand, p)
    t = p

    gt = u > t
    eq = u == t
    cnt_gt = jnp.sum(gt.astype(jnp.float32), axis=1, keepdims=True)
    pre = jnp.dot(eq.astype(jnp.bfloat16), tri_ref[...],
                  preferred_element_type=jnp.float32)
    sel = gt | (eq & (pre < (kf - cnt_gt)))
    hard = sel.astype(jnp.float32)
    sel_mask = (hard - s) + s

    out_ref[...] = x * sel_mask[:, :, None]
    mask_ref[...] = sel_mask
    ek_ref[pl.ds(i * R, R), :] = ek


@jax.jit
def kernel(token_embeddings, W1, b1, W2, b2):
    tri = jnp.triu(jnp.ones((S, S), jnp.bfloat16), k=1)
    out, mask, ek = pl.pallas_call(
        _fused_body,
        grid=(B // R,),
        in_specs=[
            pl.BlockSpec((R, S, D), lambda i: (i, 0, 0)),
            pl.BlockSpec((D, H), lambda i: (0, 0)),
            pl.BlockSpec((H,), lambda i: (0,)),
            pl.BlockSpec((H, 1), lambda i: (0, 0)),
            pl.BlockSpec((1,), lambda i: (0,)),
            pl.BlockSpec((S, S), lambda i: (0, 0)),
        ],
        out_specs=[
            pl.BlockSpec((R, S, D), lambda i: (i, 0, 0)),
            pl.BlockSpec((R, S), lambda i: (i, 0)),
            pl.BlockSpec((B, 1), lambda i: (0, 0)),
        ],
        out_shape=[
            jax.ShapeDtypeStruct((B, S, D), jnp.float32),
            jax.ShapeDtypeStruct((B, S), jnp.float32),
            jax.ShapeDtypeStruct((B, 1), jnp.float32),
        ],
    )(token_embeddings, W1, b1, W2, b2, tri)
    return out, mask, ek[:, 0]
